# Initial kernel scaffold; baseline (speedup 1.0000x reference)
import jax
import jax.numpy as jnp
from jax.experimental import pallas as pl


def kernel(h, z, x, batch, emb_z, emb_h, sake_We, sake_Web, sake_Wh, sake_Whb, sch_f1, sch_f1b, sch_lin, sch_linb, W_out, b_out, W_o1, b_o1, W_o2, b_o2):
    raise NotImplementedError("write your pallas kernel here")


# speedup vs baseline: 8.1469x; 8.1469x over previous
import jax
import jax.numpy as jnp
from jax.experimental import pallas as pl
from jax.experimental.pallas import tpu as pltpu

N = 10000
G = 100
GS = 100
HID = 128
NRBF = 50
NL = 3


def _body(zoh_ref, hoh_ref, x_ref, xT_ref, musake_ref, musch_ref,
          embz_ref, embh_ref,
          We1_ref, We2_ref, We3_ref, Web_ref,
          Wh1_ref, Wh2_ref, Whb_ref,
          f1_ref, f1b_ref, lin_ref, linb_ref,
          Wout_ref, bout_ref, Wo1_ref, bo1_ref, Wo2_ref, bo2_ref,
          out_ref):
    Z = zoh_ref[0]
    H = hoh_ref[0]
    X = x_ref[0]
    XT = xT_ref[0]

    def mm(a, b):
        return jax.lax.dot_general(a, b, (((1,), (0,)), ((), ())),
                                   precision=jax.lax.Precision.HIGHEST,
                                   preferred_element_type=jnp.float32)

    feat = jnp.concatenate([mm(Z, embz_ref[...]), mm(H, embh_ref[...])],
                           axis=1)

    d0 = X[:, 0:1] - XT[0:1, :]
    d1 = X[:, 1:2] - XT[1:2, :]
    d2 = X[:, 2:3] - XT[2:3, :]
    radial = d0 * d0 + d1 * d1 + d2 * d2
    dist = jnp.sqrt(radial)

    ii = jax.lax.broadcasted_iota(jnp.int32, (GS, GS), 0)
    jj = jax.lax.broadcasted_iota(jnp.int32, (GS, GS), 1)
    base = (ii != jj) & (radial < 16.0)
    sakew = jnp.where(base & (dist < 2.0), 1.0 / (1.0 + radial), 0.0)
    h1 = H[:, 1:2] > 0.5
    schw = (base & h1).astype(jnp.float32)

    dist3 = dist[:, :, None]
    rbf_sake = jnp.exp(-312.5 * (dist3 - musake_ref[...]) ** 2)
    rbf_sch = jnp.exp(-78.125 * (dist3 - musch_ref[...]) ** 2)

    dn = (((2,), (0,)), ((), ()))

    def dot3(a, b):
        return jax.lax.dot_general(a, b, dn,
                                   preferred_element_type=jnp.float32)

    sakew3 = sakew[:, :, None]
    schw3 = schw[:, :, None]
    JC = 50
    NCH = GS // JC
    for i in range(NL):
        A = mm(feat, We1_ref[i])
        B = mm(feat, We2_ref[i])
        agg = jnp.zeros((GS, HID), jnp.float32)
        for c in range(NCH):
            sl = slice(c * JC, (c + 1) * JC)
            R = dot3(rbf_sake[:, sl, :], We3_ref[i])
            m = jax.nn.silu(A[:, None, :] + B[sl][None, :, :] + R
                            + Web_ref[i][None])
            agg = agg + jnp.sum(m * sakew3[:, sl], axis=1)
        feat = feat + jax.nn.silu(mm(feat, Wh1_ref[i]) + mm(agg, Wh2_ref[i])
                                  + Whb_ref[i])
        agg2 = jnp.zeros((GS, HID), jnp.float32)
        for c in range(NCH):
            sl = slice(c * JC, (c + 1) * JC)
            Wf = jax.nn.softplus(dot3(rbf_sch[:, sl, :], f1_ref[i])
                                 + f1b_ref[i][None])
            agg2 = agg2 + jnp.sum(Wf * feat[None, sl, :] * schw3[:, sl],
                                  axis=1)
        feat = feat + mm(agg2, lin_ref[i]) + linb_ref[i]

    y = mm(feat, Wout_ref[...]) + bout_ref[...]
    y = mm(jax.nn.silu(mm(y, Wo1_ref[...]) + bo1_ref[...]), Wo2_ref[...]) \
        + bo2_ref[...]
    out_ref[0] = y


@jax.jit
def _run(zoh, hoh, x3, x3T, musake, musch, embz, embh,
         We1, We2, We3, Web, Wh1, Wh2, Whb,
         f1, f1b, lin, linb, Wout, bout, Wo1, bo1, Wo2, bo2):
    def full(a):
        nd = a.ndim
        return pl.BlockSpec(a.shape, lambda g, _nd=nd: (0,) * _nd)

    bcast = (musake, musch, embz, embh, We1, We2, We3, Web, Wh1, Wh2, Whb,
             f1, f1b, lin, linb, Wout, bout, Wo1, bo1, Wo2, bo2)
    specs = [
        pl.BlockSpec((1, GS, 20), lambda g: (g, 0, 0)),
        pl.BlockSpec((1, GS, 4), lambda g: (g, 0, 0)),
        pl.BlockSpec((1, GS, 3), lambda g: (g, 0, 0)),
        pl.BlockSpec((1, 3, GS), lambda g: (g, 0, 0)),
    ] + [full(a) for a in bcast]
    return pl.pallas_call(
        _body,
        grid=(G,),
        in_specs=specs,
        out_specs=pl.BlockSpec((1, GS, 1), lambda g: (g, 0, 0)),
        out_shape=jax.ShapeDtypeStruct((G, GS, 1), jnp.float32),
        compiler_params=pltpu.CompilerParams(
            dimension_semantics=("parallel",)),
    )(zoh, hoh, x3, x3T, musake, musch, embz, embh,
      We1, We2, We3, Web, Wh1, Wh2, Whb,
      f1, f1b, lin, linb, Wout, bout, Wo1, bo1, Wo2, bo2)


def kernel(h, z, x, batch, emb_z, emb_h, sake_We, sake_Web, sake_Wh,
           sake_Whb, sch_f1, sch_f1b, sch_lin, sch_linb, W_out, b_out,
           W_o1, b_o1, W_o2, b_o2):
    zoh = jax.nn.one_hot(z, 20, dtype=jnp.float32).reshape(G, GS, 20)
    hoh = jax.nn.one_hot(h, 4, dtype=jnp.float32).reshape(G, GS, 4)
    x3 = x.reshape(G, GS, 3)
    x3T = jnp.transpose(x3, (0, 2, 1))
    musake = jnp.linspace(0.0, 2.0, NRBF, dtype=jnp.float32).reshape(1, 1, NRBF)
    musch = jnp.linspace(0.0, 4.0, NRBF, dtype=jnp.float32).reshape(1, 1, NRBF)
    We1 = sake_We[:, :HID, :]
    We2 = sake_We[:, HID:2 * HID, :]
    We3 = sake_We[:, 2 * HID:, :]
    Wh1 = sake_Wh[:, :HID, :]
    Wh2 = sake_Wh[:, HID:, :]
    y = _run(zoh, hoh, x3, x3T, musake, musch, emb_z, emb_h,
             We1, We2, We3, sake_Web.reshape(NL, 1, HID),
             Wh1, Wh2, sake_Whb.reshape(NL, 1, HID),
             sch_f1, sch_f1b.reshape(NL, 1, HID),
             sch_lin, sch_linb.reshape(NL, 1, HID),
             W_out, b_out.reshape(1, HID),
             W_o1, b_o1.reshape(1, 64), W_o2, b_o2.reshape(1, 1))
    return y.reshape(N, 1)


# speedup vs baseline: 12.9339x; 1.5876x over previous
import jax
import jax.numpy as jnp
from jax.experimental import pallas as pl
from jax.experimental.pallas import tpu as pltpu

N = 10000
G = 100
GS = 100
HID = 128
NRBF = 50
NL = 3
JC = 50
NCH = GS // JC


def _body(zoh_ref, hoh_ref, x_ref, xT_ref, mucat_ref, gcat_ref,
          embz_ref, embh_ref,
          We12_ref, Wrbf_ref, Web_ref,
          Wh_ref, Whb_ref,
          f1b_ref, lin_ref, linb_ref,
          Wout_ref, bout_ref, Wo1_ref, bo1_ref, Wo2_ref, bo2_ref,
          out_ref, wfp_ref):
    Z = zoh_ref[0]
    H = hoh_ref[0]
    X = x_ref[0]
    XT = xT_ref[0]

    def mm(a, b):
        return jax.lax.dot_general(a, b, (((1,), (0,)), ((), ())),
                                   precision=jax.lax.Precision.HIGHEST,
                                   preferred_element_type=jnp.float32)

    feat = jnp.concatenate([mm(Z, embz_ref[...]), mm(H, embh_ref[...])],
                           axis=1)

    d0 = X[:, 0:1] - XT[0:1, :]
    d1 = X[:, 1:2] - XT[1:2, :]
    d2 = X[:, 2:3] - XT[2:3, :]
    radial = d0 * d0 + d1 * d1 + d2 * d2
    dist = jnp.sqrt(radial)

    ii = jax.lax.broadcasted_iota(jnp.int32, (GS, GS), 0)
    jj = jax.lax.broadcasted_iota(jnp.int32, (GS, GS), 1)
    base = (ii != jj) & (radial < 16.0)
    sakew = jnp.where(base & (dist < 2.0), 1.0 / (1.0 + radial), 0.0)
    h1 = H[:, 1:2] > 0.5
    schw = (base & h1).astype(jnp.float32)

    dist3 = dist[:, :, None]
    rbf = jnp.exp(gcat_ref[...] * (dist3 - mucat_ref[...]) ** 2)

    dn = (((2,), (0,)), ((), ()))

    def dot3(a, b):
        return jax.lax.dot_general(a, b, dn,
                                   preferred_element_type=jnp.float32)

    sakew3 = sakew[:, :, None]
    schw3 = schw[:, :, None]
    for i in range(NL):
        AB = mm(feat, We12_ref[i])
        A = AB[:, :HID] + Web_ref[i]
        B = AB[:, HID:]
        agg = jnp.zeros((GS, HID), jnp.float32)
        for c in range(NCH):
            sl = slice(c * JC, (c + 1) * JC)
            RW = dot3(rbf[:, sl, :], Wrbf_ref[i])
            m = jax.nn.silu(A[:, None, :] + B[sl][None, :, :]
                            + RW[:, :, :HID])
            agg = agg + jnp.sum(m * sakew3[:, sl], axis=1)
            wfp_ref[:, sl, :] = RW[:, :, HID:] + f1b_ref[i][None]
        feat = feat + jax.nn.silu(
            mm(jnp.concatenate([feat, agg], axis=1), Wh_ref[i])
            + Whb_ref[i])
        agg2 = jnp.zeros((GS, HID), jnp.float32)
        for c in range(NCH):
            sl = slice(c * JC, (c + 1) * JC)
            Wf = jax.nn.softplus(wfp_ref[:, sl, :])
            agg2 = agg2 + jnp.sum(Wf * feat[None, sl, :] * schw3[:, sl],
                                  axis=1)
        feat = feat + mm(agg2, lin_ref[i]) + linb_ref[i]

    y = mm(feat, Wout_ref[...]) + bout_ref[...]
    y = mm(jax.nn.silu(mm(y, Wo1_ref[...]) + bo1_ref[...]), Wo2_ref[...]) \
        + bo2_ref[...]
    out_ref[0] = y


@jax.jit
def _run(zoh, hoh, x3, x3T, mucat, gcat, embz, embh,
         We12, Wrbf, Web, Wh, Whb, f1b, lin, linb,
         Wout, bout, Wo1, bo1, Wo2, bo2):
    def full(a):
        nd = a.ndim
        return pl.BlockSpec(a.shape, lambda g, _nd=nd: (0,) * _nd)

    bcast = (mucat, gcat, embz, embh, We12, Wrbf, Web, Wh, Whb,
             f1b, lin, linb, Wout, bout, Wo1, bo1, Wo2, bo2)
    specs = [
        pl.BlockSpec((1, GS, 20), lambda g: (g, 0, 0)),
        pl.BlockSpec((1, GS, 4), lambda g: (g, 0, 0)),
        pl.BlockSpec((1, GS, 3), lambda g: (g, 0, 0)),
        pl.BlockSpec((1, 3, GS), lambda g: (g, 0, 0)),
    ] + [full(a) for a in bcast]
    return pl.pallas_call(
        _body,
        grid=(G,),
        in_specs=specs,
        out_specs=pl.BlockSpec((1, GS, 1), lambda g: (g, 0, 0)),
        out_shape=jax.ShapeDtypeStruct((G, GS, 1), jnp.float32),
        scratch_shapes=[pltpu.VMEM((GS, GS, HID), jnp.float32)],
        compiler_params=pltpu.CompilerParams(
            dimension_semantics=("parallel",)),
    )(zoh, hoh, x3, x3T, mucat, gcat, embz, embh,
      We12, Wrbf, Web, Wh, Whb, f1b, lin, linb,
      Wout, bout, Wo1, bo1, Wo2, bo2)


def kernel(h, z, x, batch, emb_z, emb_h, sake_We, sake_Web, sake_Wh,
           sake_Whb, sch_f1, sch_f1b, sch_lin, sch_linb, W_out, b_out,
           W_o1, b_o1, W_o2, b_o2):
    zoh = jax.nn.one_hot(z, 20, dtype=jnp.float32).reshape(G, GS, 20)
    hoh = jax.nn.one_hot(h, 4, dtype=jnp.float32).reshape(G, GS, 4)
    x3 = x.reshape(G, GS, 3)
    x3T = jnp.transpose(x3, (0, 2, 1))
    musake = jnp.linspace(0.0, 2.0, NRBF, dtype=jnp.float32)
    musch = jnp.linspace(0.0, 4.0, NRBF, dtype=jnp.float32)
    mucat = jnp.concatenate([musake, musch]).reshape(1, 1, 2 * NRBF)
    gcat = jnp.concatenate([jnp.full((NRBF,), -312.5, jnp.float32),
                            jnp.full((NRBF,), -78.125, jnp.float32)]
                           ).reshape(1, 1, 2 * NRBF)
    We12 = jnp.concatenate([sake_We[:, :HID, :], sake_We[:, HID:2 * HID, :]],
                           axis=2)
    We3 = sake_We[:, 2 * HID:, :]
    zblk = jnp.zeros((NL, NRBF, HID), jnp.float32)
    Wrbf = jnp.concatenate([
        jnp.concatenate([We3, zblk], axis=2),
        jnp.concatenate([zblk, sch_f1], axis=2),
    ], axis=1)
    y = _run(zoh, hoh, x3, x3T, mucat, gcat, emb_z, emb_h,
             We12, Wrbf, sake_Web.reshape(NL, 1, HID),
             sake_Wh, sake_Whb.reshape(NL, 1, HID),
             sch_f1b.reshape(NL, 1, HID),
             sch_lin, sch_linb.reshape(NL, 1, HID),
             W_out, b_out.reshape(1, HID),
             W_o1, b_o1.reshape(1, 64), W_o2, b_o2.reshape(1, 1))
    return y.reshape(N, 1)


# speedup vs baseline: 25.8058x; 1.9952x over previous
import jax
import jax.numpy as jnp
from jax.experimental import pallas as pl
from jax.experimental.pallas import tpu as pltpu

N = 10000
G = 100
GS = 100
HID = 128
NRBF = 50
NL = 3
JC = 50
NCH = GS // JC


def _body(zoh_ref, hoh_ref, x_ref, xT_ref, mucat_ref, gcat_ref,
          embz_ref, embh_ref,
          We12_ref, Wrbf_ref, Web_ref,
          Wh_ref, Whb_ref,
          f1b_ref, lin_ref, linb_ref,
          Wout_ref, bout_ref, Wo1_ref, bo1_ref, Wo2_ref, bo2_ref,
          out_ref, wfp_ref):
    Z = zoh_ref[0]
    H = hoh_ref[0]
    X = x_ref[0]
    XT = xT_ref[0]

    def mm(a, b):
        return jax.lax.dot_general(a, b, (((1,), (0,)), ((), ())),
                                   precision=jax.lax.Precision.HIGHEST,
                                   preferred_element_type=jnp.float32)

    feat = jnp.concatenate([mm(Z, embz_ref[...]), mm(H, embh_ref[...])],
                           axis=1)

    d0 = X[:, 0:1] - XT[0:1, :]
    d1 = X[:, 1:2] - XT[1:2, :]
    d2 = X[:, 2:3] - XT[2:3, :]
    radial = d0 * d0 + d1 * d1 + d2 * d2
    dist = jnp.sqrt(radial)

    ii = jax.lax.broadcasted_iota(jnp.int32, (GS, GS), 0)
    jj = jax.lax.broadcasted_iota(jnp.int32, (GS, GS), 1)
    base = (ii != jj) & (radial < 16.0)
    sakew = jnp.where(base & (dist < 2.0), 1.0 / (1.0 + radial), 0.0)
    h1f = H[:, 1:2]

    dist3 = dist[:, :, None]
    rbf = jnp.exp(gcat_ref[...] * (dist3 - mucat_ref[...]) ** 2)

    dn = (((2,), (0,)), ((), ()))

    def dot3(a, b):
        return jax.lax.dot_general(a, b, dn,
                                   preferred_element_type=jnp.float32)

    sakew3 = sakew[:, :, None]
    basew3 = base.astype(jnp.float32)[:, :, None]
    for i in range(NL):
        AB = mm(feat, We12_ref[i])
        A = AB[:, :HID] + Web_ref[i]
        B = AB[:, HID:]
        agg = jnp.zeros((GS, HID), jnp.float32)
        for c in range(NCH):
            sl = slice(c * JC, (c + 1) * JC)
            RW = dot3(rbf[sl], Wrbf_ref[i])
            m = jax.nn.silu(A[None, :, :] + B[sl][:, None, :]
                            + RW[:, :, :HID])
            agg = agg + jnp.sum(m * sakew3[sl], axis=0)
            wfp_ref[sl] = RW[:, :, HID:] + f1b_ref[i][None]
        feat = feat + jax.nn.silu(
            mm(jnp.concatenate([feat, agg], axis=1), Wh_ref[i])
            + Whb_ref[i])
        agg2 = jnp.zeros((GS, HID), jnp.float32)
        for c in range(NCH):
            sl = slice(c * JC, (c + 1) * JC)
            Wf = jax.nn.softplus(wfp_ref[sl])
            agg2 = agg2 + jnp.sum(Wf * feat[sl][:, None, :] * basew3[sl],
                                  axis=0)
        feat = feat + mm(h1f * agg2, lin_ref[i]) + linb_ref[i]

    y = mm(feat, Wout_ref[...]) + bout_ref[...]
    y = mm(jax.nn.silu(mm(y, Wo1_ref[...]) + bo1_ref[...]), Wo2_ref[...]) \
        + bo2_ref[...]
    out_ref[0] = y


@jax.jit
def _run(zoh, hoh, x3, x3T, mucat, gcat, embz, embh,
         We12, Wrbf, Web, Wh, Whb, f1b, lin, linb,
         Wout, bout, Wo1, bo1, Wo2, bo2):
    def full(a):
        nd = a.ndim
        return pl.BlockSpec(a.shape, lambda g, _nd=nd: (0,) * _nd)

    bcast = (mucat, gcat, embz, embh, We12, Wrbf, Web, Wh, Whb,
             f1b, lin, linb, Wout, bout, Wo1, bo1, Wo2, bo2)
    specs = [
        pl.BlockSpec((1, GS, 20), lambda g: (g, 0, 0)),
        pl.BlockSpec((1, GS, 4), lambda g: (g, 0, 0)),
        pl.BlockSpec((1, GS, 3), lambda g: (g, 0, 0)),
        pl.BlockSpec((1, 3, GS), lambda g: (g, 0, 0)),
    ] + [full(a) for a in bcast]
    return pl.pallas_call(
        _body,
        grid=(G,),
        in_specs=specs,
        out_specs=pl.BlockSpec((1, GS, 1), lambda g: (g, 0, 0)),
        out_shape=jax.ShapeDtypeStruct((G, GS, 1), jnp.float32),
        scratch_shapes=[pltpu.VMEM((GS, GS, HID), jnp.float32)],
        compiler_params=pltpu.CompilerParams(
            dimension_semantics=("parallel",)),
    )(zoh, hoh, x3, x3T, mucat, gcat, embz, embh,
      We12, Wrbf, Web, Wh, Whb, f1b, lin, linb,
      Wout, bout, Wo1, bo1, Wo2, bo2)


def kernel(h, z, x, batch, emb_z, emb_h, sake_We, sake_Web, sake_Wh,
           sake_Whb, sch_f1, sch_f1b, sch_lin, sch_linb, W_out, b_out,
           W_o1, b_o1, W_o2, b_o2):
    zoh = jax.nn.one_hot(z, 20, dtype=jnp.float32).reshape(G, GS, 20)
    hoh = jax.nn.one_hot(h, 4, dtype=jnp.float32).reshape(G, GS, 4)
    x3 = x.reshape(G, GS, 3)
    x3T = jnp.transpose(x3, (0, 2, 1))
    musake = jnp.linspace(0.0, 2.0, NRBF, dtype=jnp.float32)
    musch = jnp.linspace(0.0, 4.0, NRBF, dtype=jnp.float32)
    mucat = jnp.concatenate([musake, musch]).reshape(1, 1, 2 * NRBF)
    gcat = jnp.concatenate([jnp.full((NRBF,), -312.5, jnp.float32),
                            jnp.full((NRBF,), -78.125, jnp.float32)]
                           ).reshape(1, 1, 2 * NRBF)
    We12 = jnp.concatenate([sake_We[:, :HID, :], sake_We[:, HID:2 * HID, :]],
                           axis=2)
    We3 = sake_We[:, 2 * HID:, :]
    zblk = jnp.zeros((NL, NRBF, HID), jnp.float32)
    Wrbf = jnp.concatenate([
        jnp.concatenate([We3, zblk], axis=2),
        jnp.concatenate([zblk, sch_f1], axis=2),
    ], axis=1)
    y = _run(zoh, hoh, x3, x3T, mucat, gcat, emb_z, emb_h,
             We12, Wrbf, sake_Web.reshape(NL, 1, HID),
             sake_Wh, sake_Whb.reshape(NL, 1, HID),
             sch_f1b.reshape(NL, 1, HID),
             sch_lin, sch_linb.reshape(NL, 1, HID),
             W_out, b_out.reshape(1, HID),
             W_o1, b_o1.reshape(1, 64), W_o2, b_o2.reshape(1, 1))
    return y.reshape(N, 1)
